# baseline (speedup 1.0000x reference)
import functools

import jax
import jax.numpy as jnp
from jax import lax
from jax.experimental import pallas as pl
from jax.experimental.pallas import tpu as pltpu
from jax.experimental.pallas import tpu_sc as plsc

_N = 10000
_E = 320000
_F = 128
_SLOPE = 0.2

_N_PAD = 10240
_NW = 32
_CHUNK = 64
_NBUF = 4
_GRP = 32
_C0 = 160
_C1 = 160
_NCHUNKS = 16 * (_C0 + _C1)
_DEG_CHUNKS = _NCHUNKS // _NW
_E_PAD = _NCHUNKS * _CHUNK
_TILE_ROWS = _N_PAD // 16
_ROW_BLK = 512

_mesh = plsc.VectorSubcoreMesh(core_axis_name="c", subcore_axis_name="s")


@functools.partial(
    pl.kernel,
    out_type=jax.ShapeDtypeStruct((2, _N_PAD, _F), jnp.float32),
    mesh=_mesh,
    scratch_types=[
        pltpu.VMEM((_GRP, _CHUNK), jnp.int32),
        pltpu.VMEM((_GRP, 1, _CHUNK), jnp.int32),
        pltpu.VMEM((_NBUF, _CHUNK, _F), jnp.float32),
        pltpu.VMEM_SHARED((_N_PAD, _F), jnp.float32),
        [pltpu.SemaphoreType.DMA] * _NBUF,
        [pltpu.SemaphoreType.DMA] * _NBUF,
    ],
)
def _sc_aggregate(h_hbm, src_hbm, dst_hbm, out_hbm,
                  src_v, dst_v, bufs, acc, gsem, ssem):
    cid = lax.axis_index("c")
    sid = lax.axis_index("s")

    def _zrow(r, carry):
        for j in range(_F // 16):
            bufs[0, r, pl.ds(j * 16, 16)] = jnp.zeros((16,), jnp.float32)
        return carry
    lax.fori_loop(0, _CHUNK, _zrow, 0)
    for k in range(_TILE_ROWS // _CHUNK):
        pltpu.sync_copy(bufs.at[0],
                        acc.at[pl.ds(sid * _TILE_ROWS + k * _CHUNK, _CHUNK)])
    plsc.subcore_barrier()

    base = jnp.where(cid == 0, sid * _C0, 16 * _C0 + sid * _C1)
    if _C0 == _C1:
        ngrp = _C0 // _GRP
    else:
        ngrp = jnp.where(cid == 0, _C0 // _GRP, _C1 // _GRP)
    qpg = _GRP // _NBUF

    def _group(g, carry):
        pltpu.sync_copy(src_hbm.at[pl.ds(base + g * _GRP, _GRP)], src_v)
        pltpu.sync_copy(dst_hbm.at[pl.ds(base + g * _GRP, _GRP)], dst_v)
        for b in range(_NBUF):
            pltpu.async_copy(h_hbm.at[src_v.at[b]], bufs.at[b], gsem[b])

        def _round(i, c):
            j0 = i * _NBUF
            for b in range(_NBUF):
                pltpu.make_async_copy(h_hbm.at[src_v.at[0]], bufs.at[b],
                                      gsem[b]).wait()
                pltpu.async_copy(bufs.at[b], acc.at[dst_v.at[j0 + b, 0]],
                                 ssem[b], add=True)

            @pl.when(i < qpg - 1)
            def _refill():
                for b in range(_NBUF):
                    pltpu.make_async_copy(bufs.at[b], acc.at[dst_v.at[0, 0]],
                                          ssem[b]).wait()
                    pltpu.async_copy(h_hbm.at[src_v.at[j0 + _NBUF + b]],
                                     bufs.at[b], gsem[b])

            @pl.when(i == qpg - 1)
            def _drain():
                for b in range(_NBUF):
                    pltpu.make_async_copy(bufs.at[b], acc.at[dst_v.at[0, 0]],
                                          ssem[b]).wait()
            return c
        lax.fori_loop(0, qpg, _round, 0)
        return carry

    lax.fori_loop(0, ngrp, _group, 0)
    plsc.subcore_barrier()

    for k in range(_TILE_ROWS // _CHUNK):
        r = sid * _TILE_ROWS + k * _CHUNK
        pltpu.sync_copy(acc.at[pl.ds(r, _CHUNK)], bufs.at[0])
        pltpu.sync_copy(bufs.at[0], out_hbm.at[cid, pl.ds(r, _CHUNK)])


@functools.partial(
    pl.kernel,
    out_type=jax.ShapeDtypeStruct((_NW, _N_PAD), jnp.float32),
    mesh=_mesh,
    scratch_types=[
        pltpu.VMEM((_DEG_CHUNKS, 1, _CHUNK), jnp.int32),
        pltpu.VMEM((_N_PAD,), jnp.float32),
    ],
    compiler_params=pltpu.CompilerParams(needs_layout_passes=False),
)
def _sc_degree(dst_hbm, out_hbm, dst_v, acc):
    cid = lax.axis_index("c")
    sid = lax.axis_index("s")
    wid = cid * 16 + sid

    pltpu.sync_copy(dst_hbm.at[pl.ds(wid * _DEG_CHUNKS, _DEG_CHUNKS)], dst_v)

    def _z(i, carry):
        acc[pl.ds(i * 16, 16)] = jnp.zeros((16,), jnp.float32)
        return carry
    lax.fori_loop(0, _N_PAD // 16, _z, 0)

    ones16 = jnp.ones((16,), jnp.float32)

    def _edge_chunk(j, carry):
        for k in range(_CHUNK // 16):
            idx = dst_v[j, 0, pl.ds(k * 16, 16)]
            plsc.addupdate_scatter(acc, [idx], ones16)
        return carry
    lax.fori_loop(0, _DEG_CHUNKS, _edge_chunk, 0)

    pltpu.sync_copy(acc, out_hbm.at[wid])


def _mm_body(x_ref, w_ref, o_ref):
    o_ref[:] = jnp.dot(x_ref[:], w_ref[:], preferred_element_type=jnp.float32)


_mm = pl.pallas_call(
    _mm_body,
    grid=(_N_PAD // _ROW_BLK,),
    in_specs=[pl.BlockSpec((_ROW_BLK, _F), lambda i: (i, 0)),
              pl.BlockSpec((_F, _F), lambda i: (0, 0))],
    out_specs=pl.BlockSpec((_ROW_BLK, _F), lambda i: (i, 0)),
    out_shape=jax.ShapeDtypeStruct((_N_PAD, _F), jnp.float32),
)


def _prep_body(d_ref, h_ref, dinv_ref, hp_ref):
    deg = jnp.sum(d_ref[:], axis=0) + jnp.float32(1)
    dinv = lax.rsqrt(jnp.maximum(deg, jnp.float32(1)))
    dinv_ref[:] = dinv[:, None]
    hp_ref[:] = h_ref[:] * dinv[:, None]


_prep = pl.pallas_call(
    _prep_body,
    grid=(_N_PAD // _ROW_BLK,),
    in_specs=[pl.BlockSpec((_NW, _ROW_BLK), lambda i: (0, i)),
              pl.BlockSpec((_ROW_BLK, _F), lambda i: (i, 0))],
    out_specs=[pl.BlockSpec((_ROW_BLK, 1), lambda i: (i, 0)),
               pl.BlockSpec((_ROW_BLK, _F), lambda i: (i, 0))],
    out_shape=[jax.ShapeDtypeStruct((_N_PAD, 1), jnp.float32),
               jax.ShapeDtypeStruct((_N_PAD, _F), jnp.float32)],
)


def _combine_body(p0_ref, p1_ref, hp_ref, dinv_ref, b_ref, w_ref, o_ref):
    dinv = dinv_ref[:]
    agg = (p0_ref[:] + p1_ref[:] + hp_ref[:]) * dinv + b_ref[:]
    act = jnp.where(agg >= 0, agg, _SLOPE * agg)
    o_ref[:] = jnp.dot(act, w_ref[:], preferred_element_type=jnp.float32) * dinv


_combine = pl.pallas_call(
    _combine_body,
    grid=(_N_PAD // _ROW_BLK,),
    in_specs=[pl.BlockSpec((_ROW_BLK, _F), lambda i: (i, 0)),
              pl.BlockSpec((_ROW_BLK, _F), lambda i: (i, 0)),
              pl.BlockSpec((_ROW_BLK, _F), lambda i: (i, 0)),
              pl.BlockSpec((_ROW_BLK, 1), lambda i: (i, 0)),
              pl.BlockSpec((1, _F), lambda i: (0, 0)),
              pl.BlockSpec((_F, _F), lambda i: (0, 0))],
    out_specs=pl.BlockSpec((_ROW_BLK, _F), lambda i: (i, 0)),
    out_shape=jax.ShapeDtypeStruct((_N_PAD, _F), jnp.float32),
)


def _final_body(p0_ref, p1_ref, hp_ref, dinv_ref, b_ref, o_ref):
    o_ref[:] = (p0_ref[:] + p1_ref[:] + hp_ref[:]) * dinv_ref[:] + b_ref[:]


_final = pl.pallas_call(
    _final_body,
    grid=(_N_PAD // _ROW_BLK,),
    in_specs=[pl.BlockSpec((_ROW_BLK, _F), lambda i: (i, 0)),
              pl.BlockSpec((_ROW_BLK, _F), lambda i: (i, 0)),
              pl.BlockSpec((_ROW_BLK, _F), lambda i: (i, 0)),
              pl.BlockSpec((_ROW_BLK, 1), lambda i: (i, 0)),
              pl.BlockSpec((1, _F), lambda i: (0, 0))],
    out_specs=pl.BlockSpec((_ROW_BLK, _F), lambda i: (i, 0)),
    out_shape=jax.ShapeDtypeStruct((_N_PAD, _F), jnp.float32),
)


def kernel(x, edge_index, W1, b1, W2, b2, W3, b3, W4, b4, W5, b5, W6, b6):
    src = edge_index[0].astype(jnp.int32)
    dst = edge_index[1].astype(jnp.int32)
    pad = _E_PAD - _E
    src_r = jnp.concatenate(
        [src, jnp.zeros((pad,), jnp.int32)]).reshape(_NCHUNKS, _CHUNK)
    dst_r = jnp.concatenate(
        [dst, jnp.full((pad,), _N, jnp.int32)]).reshape(_NCHUNKS, 1, _CHUNK)
    x_p = jnp.pad(x, ((0, _N_PAD - _N), (0, 0)))

    degp = _sc_degree(dst_r)
    h1 = _mm(x_p, W1)
    dinv, hp = _prep(degp, h1)

    for W, b in zip((W2, W3, W4, W5, W6), (b1, b2, b3, b4, b5)):
        p = _sc_aggregate(hp, src_r, dst_r)
        hp = _combine(p[0], p[1], hp, dinv, b.reshape(1, _F), W)

    p = _sc_aggregate(hp, src_r, dst_r)
    out = _final(p[0], p[1], hp, dinv, b6.reshape(1, _F))
    return out[:_N]


# speedup vs baseline: 1.2016x; 1.2016x over previous
import functools

import jax
import jax.numpy as jnp
from jax import lax
from jax.experimental import pallas as pl
from jax.experimental.pallas import tpu as pltpu
from jax.experimental.pallas import tpu_sc as plsc

_N = 10000
_E = 320000
_F = 128
_SLOPE = 0.2

_N_PAD = 10240
_NW = 32
_CHUNK = 128
_CHUNKS_PER_W = 80
_GRP = 16
_NGRP = _CHUNKS_PER_W // _GRP
_E_PAD = _NW * _CHUNKS_PER_W * _CHUNK
_TILE_ROWS = _N_PAD // 16
_ROW_BLK = 512

_mesh = plsc.VectorSubcoreMesh(core_axis_name="c", subcore_axis_name="s")


@functools.partial(
    pl.kernel,
    out_type=jax.ShapeDtypeStruct((2, _N_PAD, _F), jnp.float32),
    mesh=_mesh,
    scratch_types=[
        pltpu.VMEM((_GRP, _CHUNK), jnp.int32),
        pltpu.VMEM((_GRP, 1, _CHUNK), jnp.int32),
        pltpu.VMEM((_CHUNK, _F), jnp.float32),
        pltpu.VMEM((_CHUNK, _F), jnp.float32),
        pltpu.VMEM_SHARED((_N_PAD, _F), jnp.float32),
        pltpu.SemaphoreType.DMA,
        pltpu.SemaphoreType.DMA,
    ],
)
def _sc_aggregate(h_hbm, src_hbm, dst_hbm, out_hbm,
                  src_v, dst_v, buf0, buf1, acc, sem0, sem1):
    cid = lax.axis_index("c")
    sid = lax.axis_index("s")
    wid = cid * 16 + sid

    def _zrow(r, carry):
        for j in range(_F // 16):
            buf0[r, pl.ds(j * 16, 16)] = jnp.zeros((16,), jnp.float32)
        return carry
    lax.fori_loop(0, _CHUNK, _zrow, 0)
    for k in range(_TILE_ROWS // _CHUNK):
        pltpu.sync_copy(buf0, acc.at[pl.ds(sid * _TILE_ROWS + k * _CHUNK, _CHUNK)])
    plsc.subcore_barrier()

    def _group(g, carry):
        pltpu.sync_copy(src_hbm.at[wid, pl.ds(g * _GRP, _GRP)], src_v)
        pltpu.sync_copy(dst_hbm.at[wid, pl.ds(g * _GRP, _GRP)], dst_v)
        pltpu.async_copy(h_hbm.at[src_v.at[0]], buf0, sem0)

        def _pair(i, c):
            j0 = 2 * i
            pltpu.async_copy(h_hbm.at[src_v.at[j0 + 1]], buf1, sem1)
            pltpu.make_async_copy(h_hbm.at[src_v.at[0]], buf0, sem0).wait()
            pltpu.sync_copy(buf0, acc.at[dst_v.at[j0, 0]], add=True)
            pltpu.async_copy(h_hbm.at[src_v.at[jnp.minimum(j0 + 2, _GRP - 1)]],
                             buf0, sem0)
            pltpu.make_async_copy(h_hbm.at[src_v.at[0]], buf1, sem1).wait()
            pltpu.sync_copy(buf1, acc.at[dst_v.at[j0 + 1, 0]], add=True)
            return c
        lax.fori_loop(0, _GRP // 2, _pair, 0)
        pltpu.make_async_copy(h_hbm.at[src_v.at[0]], buf0, sem0).wait()
        return carry
    lax.fori_loop(0, _NGRP, _group, 0)
    plsc.subcore_barrier()

    for k in range(_TILE_ROWS // _CHUNK):
        r = sid * _TILE_ROWS + k * _CHUNK
        pltpu.sync_copy(acc.at[pl.ds(r, _CHUNK)], buf0)
        pltpu.sync_copy(buf0, out_hbm.at[cid, pl.ds(r, _CHUNK)])


@functools.partial(
    pl.kernel,
    out_type=jax.ShapeDtypeStruct((_NW, _N_PAD), jnp.float32),
    mesh=_mesh,
    scratch_types=[
        pltpu.VMEM((_CHUNKS_PER_W, 1, _CHUNK), jnp.int32),
        pltpu.VMEM((_N_PAD,), jnp.float32),
    ],
    compiler_params=pltpu.CompilerParams(needs_layout_passes=False),
)
def _sc_degree(dst_hbm, out_hbm, dst_v, acc):
    cid = lax.axis_index("c")
    sid = lax.axis_index("s")
    wid = cid * 16 + sid

    pltpu.sync_copy(dst_hbm.at[wid], dst_v)

    def _z(i, carry):
        acc[pl.ds(i * 16, 16)] = jnp.zeros((16,), jnp.float32)
        return carry
    lax.fori_loop(0, _N_PAD // 16, _z, 0)

    ones16 = jnp.ones((16,), jnp.float32)

    def _edge_chunk(j, carry):
        for k in range(_CHUNK // 16):
            idx = dst_v[j, 0, pl.ds(k * 16, 16)]
            plsc.addupdate_scatter(acc, [idx], ones16)
        return carry
    lax.fori_loop(0, _CHUNKS_PER_W, _edge_chunk, 0)

    pltpu.sync_copy(acc, out_hbm.at[wid])


def _mm_body(x_ref, w_ref, o_ref):
    o_ref[:] = jnp.dot(x_ref[:], w_ref[:], preferred_element_type=jnp.float32)


_mm = pl.pallas_call(
    _mm_body,
    grid=(_N_PAD // _ROW_BLK,),
    in_specs=[pl.BlockSpec((_ROW_BLK, _F), lambda i: (i, 0)),
              pl.BlockSpec((_F, _F), lambda i: (0, 0))],
    out_specs=pl.BlockSpec((_ROW_BLK, _F), lambda i: (i, 0)),
    out_shape=jax.ShapeDtypeStruct((_N_PAD, _F), jnp.float32),
)


def _prep_body(d_ref, h_ref, dinv_ref, hp_ref):
    deg = jnp.sum(d_ref[:], axis=0) + jnp.float32(1)
    dinv = lax.rsqrt(jnp.maximum(deg, jnp.float32(1)))
    dinv_ref[:] = dinv[:, None]
    hp_ref[:] = h_ref[:] * dinv[:, None]


_prep = pl.pallas_call(
    _prep_body,
    grid=(_N_PAD // _ROW_BLK,),
    in_specs=[pl.BlockSpec((_NW, _ROW_BLK), lambda i: (0, i)),
              pl.BlockSpec((_ROW_BLK, _F), lambda i: (i, 0))],
    out_specs=[pl.BlockSpec((_ROW_BLK, 1), lambda i: (i, 0)),
               pl.BlockSpec((_ROW_BLK, _F), lambda i: (i, 0))],
    out_shape=[jax.ShapeDtypeStruct((_N_PAD, 1), jnp.float32),
               jax.ShapeDtypeStruct((_N_PAD, _F), jnp.float32)],
)


def _combine_body(p0_ref, p1_ref, hp_ref, dinv_ref, b_ref, w_ref, o_ref):
    dinv = dinv_ref[:]
    agg = (p0_ref[:] + p1_ref[:] + hp_ref[:]) * dinv + b_ref[:]
    act = jnp.where(agg >= 0, agg, _SLOPE * agg)
    o_ref[:] = jnp.dot(act, w_ref[:], preferred_element_type=jnp.float32) * dinv


_combine = pl.pallas_call(
    _combine_body,
    grid=(_N_PAD // _ROW_BLK,),
    in_specs=[pl.BlockSpec((_ROW_BLK, _F), lambda i: (i, 0)),
              pl.BlockSpec((_ROW_BLK, _F), lambda i: (i, 0)),
              pl.BlockSpec((_ROW_BLK, _F), lambda i: (i, 0)),
              pl.BlockSpec((_ROW_BLK, 1), lambda i: (i, 0)),
              pl.BlockSpec((1, _F), lambda i: (0, 0)),
              pl.BlockSpec((_F, _F), lambda i: (0, 0))],
    out_specs=pl.BlockSpec((_ROW_BLK, _F), lambda i: (i, 0)),
    out_shape=jax.ShapeDtypeStruct((_N_PAD, _F), jnp.float32),
)


def _final_body(p0_ref, p1_ref, hp_ref, dinv_ref, b_ref, o_ref):
    o_ref[:] = (p0_ref[:] + p1_ref[:] + hp_ref[:]) * dinv_ref[:] + b_ref[:]


_final = pl.pallas_call(
    _final_body,
    grid=(_N_PAD // _ROW_BLK,),
    in_specs=[pl.BlockSpec((_ROW_BLK, _F), lambda i: (i, 0)),
              pl.BlockSpec((_ROW_BLK, _F), lambda i: (i, 0)),
              pl.BlockSpec((_ROW_BLK, _F), lambda i: (i, 0)),
              pl.BlockSpec((_ROW_BLK, 1), lambda i: (i, 0)),
              pl.BlockSpec((1, _F), lambda i: (0, 0))],
    out_specs=pl.BlockSpec((_ROW_BLK, _F), lambda i: (i, 0)),
    out_shape=jax.ShapeDtypeStruct((_N_PAD, _F), jnp.float32),
)


def kernel(x, edge_index, W1, b1, W2, b2, W3, b3, W4, b4, W5, b5, W6, b6):
    src = edge_index[0].astype(jnp.int32)
    dst = edge_index[1].astype(jnp.int32)
    pad = _E_PAD - _E
    src_r = jnp.concatenate(
        [src, jnp.zeros((pad,), jnp.int32)]).reshape(_NW, _CHUNKS_PER_W, _CHUNK)
    dst_r = jnp.concatenate(
        [dst, jnp.full((pad,), _N, jnp.int32)]).reshape(_NW, _CHUNKS_PER_W, 1, _CHUNK)
    x_p = jnp.pad(x, ((0, _N_PAD - _N), (0, 0)))

    degp = _sc_degree(dst_r)
    h1 = _mm(x_p, W1)
    dinv, hp = _prep(degp, h1)

    for W, b in zip((W2, W3, W4, W5, W6), (b1, b2, b3, b4, b5)):
        p = _sc_aggregate(hp, src_r, dst_r)
        hp = _combine(p[0], p[1], hp, dinv, b.reshape(1, _F), W)

    p = _sc_aggregate(hp, src_r, dst_r)
    out = _final(p[0], p[1], hp, dinv, b6.reshape(1, _F))
    return out[:_N]


# speedup vs baseline: 1.3131x; 1.0928x over previous
import functools

import jax
import jax.numpy as jnp
from jax import lax
from jax.experimental import pallas as pl
from jax.experimental.pallas import tpu as pltpu
from jax.experimental.pallas import tpu_sc as plsc

_N = 10000
_E = 320000
_F = 128
_SLOPE = 0.2

_N_PAD = 10240
_NW = 32
_CHUNK = 128
_CHUNKS_PER_W = 80
_GRP = 16
_NGRP = _CHUNKS_PER_W // _GRP
_XTRA = 2
_E_PAD = _NW * _CHUNKS_PER_W * _CHUNK
_TILE_ROWS = _N_PAD // 16
_ROW_BLK = 512

_mesh = plsc.VectorSubcoreMesh(core_axis_name="c", subcore_axis_name="s")


@functools.partial(
    pl.kernel,
    out_type=jax.ShapeDtypeStruct((2, _N_PAD, _F), jnp.float32),
    mesh=_mesh,
    scratch_types=[
        pltpu.VMEM((_GRP, _CHUNK), jnp.int32),
        pltpu.VMEM((_GRP, 1, _CHUNK), jnp.int32),
        pltpu.VMEM((_CHUNK, _F), jnp.float32),
        pltpu.VMEM((_CHUNK, _F), jnp.float32),
        pltpu.VMEM_SHARED((_N_PAD, _F), jnp.float32),
        pltpu.SemaphoreType.DMA,
        pltpu.SemaphoreType.DMA,
    ],
)
def _sc_aggregate(h_hbm, src_hbm, dst_hbm, out_hbm,
                  src_v, dst_v, buf0, buf1, acc, sem0, sem1):
    cid = lax.axis_index("c")
    sid = lax.axis_index("s")
    wid = cid * 16 + sid

    def _zrow(r, carry):
        for j in range(_F // 16):
            buf0[r, pl.ds(j * 16, 16)] = jnp.zeros((16,), jnp.float32)
        return carry
    lax.fori_loop(0, _CHUNK, _zrow, 0)
    for k in range(_TILE_ROWS // _CHUNK):
        pltpu.sync_copy(buf0, acc.at[pl.ds(sid * _TILE_ROWS + k * _CHUNK, _CHUNK)])
    plsc.subcore_barrier()

    ngrp = jnp.where(cid == 0, _NGRP + _XTRA, _NGRP - _XTRA)

    def _group(g, carry):
        own = jnp.logical_or(cid != 0, g < _NGRP)
        row = jnp.where(own, wid, wid + 16)
        gg = jnp.where(cid == 0, jnp.where(g < _NGRP, g, g - _NGRP), g + _XTRA)
        pltpu.sync_copy(src_hbm.at[row, pl.ds(gg * _GRP, _GRP)], src_v)
        pltpu.sync_copy(dst_hbm.at[row, pl.ds(gg * _GRP, _GRP)], dst_v)
        pltpu.async_copy(h_hbm.at[src_v.at[0]], buf0, sem0)

        def _pair(i, c):
            j0 = 2 * i
            pltpu.async_copy(h_hbm.at[src_v.at[j0 + 1]], buf1, sem1)
            pltpu.make_async_copy(h_hbm.at[src_v.at[0]], buf0, sem0).wait()
            pltpu.sync_copy(buf0, acc.at[dst_v.at[j0, 0]], add=True)
            pltpu.async_copy(h_hbm.at[src_v.at[jnp.minimum(j0 + 2, _GRP - 1)]],
                             buf0, sem0)
            pltpu.make_async_copy(h_hbm.at[src_v.at[0]], buf1, sem1).wait()
            pltpu.sync_copy(buf1, acc.at[dst_v.at[j0 + 1, 0]], add=True)
            return c
        lax.fori_loop(0, _GRP // 2, _pair, 0)
        pltpu.make_async_copy(h_hbm.at[src_v.at[0]], buf0, sem0).wait()
        return carry
    lax.fori_loop(0, ngrp, _group, 0)
    plsc.subcore_barrier()

    for k in range(_TILE_ROWS // _CHUNK):
        r = sid * _TILE_ROWS + k * _CHUNK
        pltpu.sync_copy(acc.at[pl.ds(r, _CHUNK)], buf0)
        pltpu.sync_copy(buf0, out_hbm.at[cid, pl.ds(r, _CHUNK)])


@functools.partial(
    pl.kernel,
    out_type=jax.ShapeDtypeStruct((_NW, _N_PAD), jnp.float32),
    mesh=_mesh,
    scratch_types=[
        pltpu.VMEM((_CHUNKS_PER_W, 1, _CHUNK), jnp.int32),
        pltpu.VMEM((_N_PAD,), jnp.float32),
    ],
    compiler_params=pltpu.CompilerParams(needs_layout_passes=False),
)
def _sc_degree(dst_hbm, out_hbm, dst_v, acc):
    cid = lax.axis_index("c")
    sid = lax.axis_index("s")
    wid = cid * 16 + sid

    pltpu.sync_copy(dst_hbm.at[wid], dst_v)

    def _z(i, carry):
        acc[pl.ds(i * 16, 16)] = jnp.zeros((16,), jnp.float32)
        return carry
    lax.fori_loop(0, _N_PAD // 16, _z, 0)

    ones16 = jnp.ones((16,), jnp.float32)

    def _edge_chunk(j, carry):
        for k in range(_CHUNK // 16):
            idx = dst_v[j, 0, pl.ds(k * 16, 16)]
            plsc.addupdate_scatter(acc, [idx], ones16)
        return carry
    lax.fori_loop(0, _CHUNKS_PER_W, _edge_chunk, 0)

    pltpu.sync_copy(acc, out_hbm.at[wid])


def _mm_body(x_ref, w_ref, o_ref):
    o_ref[:] = jnp.dot(x_ref[:], w_ref[:], preferred_element_type=jnp.float32)


_mm = pl.pallas_call(
    _mm_body,
    grid=(_N_PAD // _ROW_BLK,),
    in_specs=[pl.BlockSpec((_ROW_BLK, _F), lambda i: (i, 0)),
              pl.BlockSpec((_F, _F), lambda i: (0, 0))],
    out_specs=pl.BlockSpec((_ROW_BLK, _F), lambda i: (i, 0)),
    out_shape=jax.ShapeDtypeStruct((_N_PAD, _F), jnp.float32),
)


def _prep_body(d_ref, h_ref, dinv_ref, hp_ref):
    deg = jnp.sum(d_ref[:], axis=0) + jnp.float32(1)
    dinv = lax.rsqrt(jnp.maximum(deg, jnp.float32(1)))
    dinv_ref[:] = dinv[:, None]
    hp_ref[:] = h_ref[:] * dinv[:, None]


_prep = pl.pallas_call(
    _prep_body,
    grid=(_N_PAD // _ROW_BLK,),
    in_specs=[pl.BlockSpec((_NW, _ROW_BLK), lambda i: (0, i)),
              pl.BlockSpec((_ROW_BLK, _F), lambda i: (i, 0))],
    out_specs=[pl.BlockSpec((_ROW_BLK, 1), lambda i: (i, 0)),
               pl.BlockSpec((_ROW_BLK, _F), lambda i: (i, 0))],
    out_shape=[jax.ShapeDtypeStruct((_N_PAD, 1), jnp.float32),
               jax.ShapeDtypeStruct((_N_PAD, _F), jnp.float32)],
)


def _combine_body(p0_ref, p1_ref, hp_ref, dinv_ref, b_ref, w_ref, o_ref):
    dinv = dinv_ref[:]
    agg = (p0_ref[:] + p1_ref[:] + hp_ref[:]) * dinv + b_ref[:]
    act = jnp.where(agg >= 0, agg, _SLOPE * agg)
    o_ref[:] = jnp.dot(act, w_ref[:], preferred_element_type=jnp.float32) * dinv


_combine = pl.pallas_call(
    _combine_body,
    grid=(_N_PAD // _ROW_BLK,),
    in_specs=[pl.BlockSpec((_ROW_BLK, _F), lambda i: (i, 0)),
              pl.BlockSpec((_ROW_BLK, _F), lambda i: (i, 0)),
              pl.BlockSpec((_ROW_BLK, _F), lambda i: (i, 0)),
              pl.BlockSpec((_ROW_BLK, 1), lambda i: (i, 0)),
              pl.BlockSpec((1, _F), lambda i: (0, 0)),
              pl.BlockSpec((_F, _F), lambda i: (0, 0))],
    out_specs=pl.BlockSpec((_ROW_BLK, _F), lambda i: (i, 0)),
    out_shape=jax.ShapeDtypeStruct((_N_PAD, _F), jnp.float32),
)


def _final_body(p0_ref, p1_ref, hp_ref, dinv_ref, b_ref, o_ref):
    o_ref[:] = (p0_ref[:] + p1_ref[:] + hp_ref[:]) * dinv_ref[:] + b_ref[:]


_final = pl.pallas_call(
    _final_body,
    grid=(_N_PAD // _ROW_BLK,),
    in_specs=[pl.BlockSpec((_ROW_BLK, _F), lambda i: (i, 0)),
              pl.BlockSpec((_ROW_BLK, _F), lambda i: (i, 0)),
              pl.BlockSpec((_ROW_BLK, _F), lambda i: (i, 0)),
              pl.BlockSpec((_ROW_BLK, 1), lambda i: (i, 0)),
              pl.BlockSpec((1, _F), lambda i: (0, 0))],
    out_specs=pl.BlockSpec((_ROW_BLK, _F), lambda i: (i, 0)),
    out_shape=jax.ShapeDtypeStruct((_N_PAD, _F), jnp.float32),
)


def kernel(x, edge_index, W1, b1, W2, b2, W3, b3, W4, b4, W5, b5, W6, b6):
    src = edge_index[0].astype(jnp.int32)
    dst = edge_index[1].astype(jnp.int32)
    pad = _E_PAD - _E
    src_r = jnp.concatenate(
        [src, jnp.zeros((pad,), jnp.int32)]).reshape(_NW, _CHUNKS_PER_W, _CHUNK)
    dst_r = jnp.concatenate(
        [dst, jnp.full((pad,), _N, jnp.int32)]).reshape(_NW, _CHUNKS_PER_W, 1, _CHUNK)
    x_p = jnp.pad(x, ((0, _N_PAD - _N), (0, 0)))

    degp = _sc_degree(dst_r)
    h1 = _mm(x_p, W1)
    dinv, hp = _prep(degp, h1)

    for W, b in zip((W2, W3, W4, W5, W6), (b1, b2, b3, b4, b5)):
        p = _sc_aggregate(hp, src_r, dst_r)
        hp = _combine(p[0], p[1], hp, dinv, b.reshape(1, _F), W)

    p = _sc_aggregate(hp, src_r, dst_r)
    out = _final(p[0], p[1], hp, dinv, b6.reshape(1, _F))
    return out[:_N]
